# Initial kernel scaffold; baseline (speedup 1.0000x reference)
import jax
import jax.numpy as jnp
from jax.experimental import pallas as pl


def kernel(emissions, transitions, head_transitions, last_transitions, lengths):
    raise NotImplementedError("write your pallas kernel here")


# speedup vs baseline: 18.0799x; 18.0799x over previous
import functools

import jax
import jax.numpy as jnp
from jax.experimental import pallas as pl
from jax.experimental.pallas import tpu as pltpu

CT = 512
NORM = 4


def _crf_fwd(len_ref, em_ref, t_ref, h_ref, l_ref, out_ref,
             eem_ref, a_ref, acc_ref, z_ref):
    i = pl.program_id(0)
    nchunks = pl.num_programs(0)

    E = jnp.exp(t_ref[...])
    lastE = jnp.exp(l_ref[...])
    lengths = len_ref[...]

    eem_ref[...] = jnp.exp(em_ref[...])

    def steps(a, acc, z, base, ks):
        for k in ks:
            t = base + k
            d = eem_ref[t]
            q = jnp.dot(a, E, preferred_element_type=jnp.float32) * d
            zs = jnp.sum(q * lastE, axis=1, keepdims=True)
            zc = acc + jnp.log(zs)
            gt = i * CT + t
            z = jnp.where(lengths == gt + 1, zc, z)
            a = q
        s = jnp.sum(a, axis=1, keepdims=True)
        acc = acc + jnp.log(s)
        a = a / s
        return a, acc, z

    def group_body(g, carry):
        a, acc, z = carry
        return steps(a, acc, z, g * NORM, list(range(NORM)))

    @pl.when(i == 0)
    def _first_chunk():
        eh = jnp.exp(h_ref[...])
        a0 = eh * eem_ref[0]
        acc0 = jnp.zeros_like(acc_ref)
        z0 = jnp.log(jnp.sum(a0 * lastE, axis=1, keepdims=True))
        z = jnp.where(lengths == 1, z0, jnp.zeros_like(z_ref))
        a, acc, z = steps(a0, acc0, z, 0, list(range(1, NORM)))
        a, acc, z = jax.lax.fori_loop(1, CT // NORM, group_body, (a, acc, z))
        a_ref[...], acc_ref[...], z_ref[...] = a, acc, z

    @pl.when(i > 0)
    def _rest_chunks():
        carry = (a_ref[...], acc_ref[...], z_ref[...])
        a, acc, z = jax.lax.fori_loop(0, CT // NORM, group_body, carry)
        a_ref[...], acc_ref[...], z_ref[...] = a, acc, z

    @pl.when(i == nchunks - 1)
    def _emit():
        out_ref[...] = z_ref[...]


@functools.partial(jax.jit, static_argnames=("interpret",))
def kernel(emissions, transitions, head_transitions, last_transitions,
           lengths, interpret=False):
    B, T, K = emissions.shape
    em_t = jnp.transpose(emissions, (1, 0, 2))
    lengths2 = jnp.maximum(lengths, 1).reshape(B, 1)
    head2 = head_transitions.reshape(1, K)
    last2 = last_transitions.reshape(1, K)

    nchunks = T // CT
    out = pl.pallas_call(
        _crf_fwd,
        grid=(nchunks,),
        in_specs=[
            pl.BlockSpec((B, 1), lambda i: (0, 0)),
            pl.BlockSpec((CT, B, K), lambda i: (i, 0, 0)),
            pl.BlockSpec((K, K), lambda i: (0, 0)),
            pl.BlockSpec((1, K), lambda i: (0, 0)),
            pl.BlockSpec((1, K), lambda i: (0, 0)),
        ],
        out_specs=pl.BlockSpec((B, 1), lambda i: (0, 0)),
        out_shape=jax.ShapeDtypeStruct((B, 1), jnp.float32),
        scratch_shapes=[
            pltpu.VMEM((CT, B, K), jnp.float32),
            pltpu.VMEM((B, K), jnp.float32),
            pltpu.VMEM((B, 1), jnp.float32),
            pltpu.VMEM((B, 1), jnp.float32),
        ],
        compiler_params=pltpu.CompilerParams(
            dimension_semantics=("arbitrary",),
        ),
        interpret=interpret,
    )(lengths2, em_t, transitions, head2, last2)
    return out.reshape(B)
